# baseline (speedup 1.0000x reference)
import functools

import jax
import jax.numpy as jnp
from jax import lax
from jax.experimental import pallas as pl
from jax.experimental.pallas import tpu as pltpu, tpu_sc as plsc

_RANK = 64
_BATCH = 16384
_NW = 32
_BPW = _BATCH // _NW
_NCH = 4
_CH = _BPW // _NCH
_L = 16


def _dpr_body(users_hbm, items_hbm, utab_hbm, itab_hbm, wm_hbm, wlv_hbm,
              bv_hbm, mean_hbm, std_hbm, logvar_hbm,
              idx_u, idx_i, u_rows, i_rows,
              mean_v, std_v, logvar_v, w_v, b_v, sem):
    wid = lax.axis_index("s") * 2 + lax.axis_index("c")
    base = wid * _BPW

    for j in range(_NCH):
        pltpu.sync_copy(users_hbm.at[pl.ds(base + j * _CH, _CH)], idx_u.at[j])
        pltpu.sync_copy(items_hbm.at[pl.ds(base + j * _CH, _CH)], idx_i.at[j])
    pltpu.sync_copy(wm_hbm, w_v.at[0])
    pltpu.sync_copy(wlv_hbm, w_v.at[1])
    pltpu.sync_copy(bv_hbm, b_v)

    copies = []
    for j in range(_NCH):
        copies.append(pltpu.async_copy(
            utab_hbm.at[idx_u.at[j]], u_rows.at[pl.ds(j * _CH, _CH)], sem))
        copies.append(pltpu.async_copy(
            itab_hbm.at[idx_i.at[j]], i_rows.at[pl.ds(j * _CH, _CH)], sem))
    for c in copies:
        c.wait()

    wm = [w_v[0, pl.ds(k * _L, _L)] for k in range(_RANK // _L)]
    wlv = [w_v[1, pl.ds(k * _L, _L)] for k in range(_RANK // _L)]

    bm = b_v[0, pl.ds(0, _L)]
    blv = b_v[1, pl.ds(0, _L)]
    lane = lax.iota(jnp.int32, _L)
    zero = jnp.zeros((_L,), jnp.float32)

    def blk_step(blk, _):
        r0 = blk * _L
        accm = zero
        acclv = zero
        for r in range(_L):
            b = r0 + r
            am = None
            alv = None
            for k in range(_RANK // _L):
                u = u_rows[b, pl.ds(k * _L, _L)]
                it = i_rows[b, pl.ds(k * _L, _L)]
                inter = u * it
                tm = inter * wm[k]
                tlv = inter * wlv[k]
                am = tm if am is None else am + tm
                alv = tlv if alv is None else alv + tlv
            sel = lane == r
            accm = jnp.where(sel, jnp.sum(am), accm)
            acclv = jnp.where(sel, jnp.sum(alv), acclv)
        lv = acclv + blv
        mean_v[pl.ds(r0, _L)] = accm + bm
        logvar_v[pl.ds(r0, _L)] = lv
        std_v[pl.ds(r0, _L)] = jnp.exp(0.5 * lv)
        return _

    lax.fori_loop(0, _BPW // _L, blk_step, 0, unroll=2)

    pltpu.sync_copy(mean_v, mean_hbm.at[pl.ds(base, _BPW)])
    pltpu.sync_copy(std_v, std_hbm.at[pl.ds(base, _BPW)])
    pltpu.sync_copy(logvar_v, logvar_hbm.at[pl.ds(base, _BPW)])


@jax.jit
def _dpr(users, items, user_table, item_table, wm, wlv, bv):
    mesh = plsc.VectorSubcoreMesh(core_axis_name="c", subcore_axis_name="s")
    out = jax.ShapeDtypeStruct((_BATCH,), jnp.float32)
    f = pl.kernel(
        _dpr_body,
        out_type=(out, out, out),
        mesh=mesh,
        scratch_types=[
            pltpu.VMEM((_NCH, _CH), jnp.int32),
            pltpu.VMEM((_NCH, _CH), jnp.int32),
            pltpu.VMEM((_BPW, _RANK), jnp.float32),
            pltpu.VMEM((_BPW, _RANK), jnp.float32),
            pltpu.VMEM((_BPW,), jnp.float32),
            pltpu.VMEM((_BPW,), jnp.float32),
            pltpu.VMEM((_BPW,), jnp.float32),
            pltpu.VMEM((2, _RANK), jnp.float32),
            pltpu.VMEM((2, _L), jnp.float32),
            pltpu.SemaphoreType.DMA,
        ],
        compiler_params=pltpu.CompilerParams(
            needs_layout_passes=False, use_tc_tiling_on_sc=False),
    )
    return f(users, items, user_table, item_table, wm, wlv, bv)


def kernel(users, items, user_table, item_table, W_mean, b_mean, W_logvar,
           b_logvar):
    wm = W_mean.reshape(_RANK)
    wlv = W_logvar.reshape(_RANK)
    bv = jnp.stack([jnp.full((_L,), b_mean[0], jnp.float32),
                    jnp.full((_L,), b_logvar[0], jnp.float32)])
    mean, std, logvar = _dpr(users, items, user_table, item_table, wm, wlv, bv)
    return (mean, std, logvar)


# speedup vs baseline: 2.1887x; 2.1887x over previous
import jax
import jax.numpy as jnp
from jax import lax
from jax.experimental import pallas as pl
from jax.experimental.pallas import tpu as pltpu, tpu_sc as plsc

_RANK = 64
_BATCH = 16384
_NW = 32
_BPW = _BATCH // _NW
_CH = 32
_NCH = _BPW // _CH
_L = 16


def _dpr_body(users_hbm, items_hbm, utab_hbm, itab_hbm, w_hbm, b_hbm,
              mean_hbm, std_hbm, logvar_hbm,
              uidx, iidx, ubuf, ibuf,
              mean_v, std_v, logvar_v, w_v, b_v, sem):
    wid = lax.axis_index("s") * 2 + lax.axis_index("c")
    base = wid * _BPW

    pltpu.sync_copy(users_hbm.at[pl.ds(base, _BPW)], uidx)
    pltpu.sync_copy(items_hbm.at[pl.ds(base, _BPW)], iidx)
    pltpu.sync_copy(w_hbm, w_v)
    pltpu.sync_copy(b_hbm, b_v)

    wm = [w_v[0, pl.ds(k * _L, _L)] for k in range(_RANK // _L)]
    wlv = [w_v[1, pl.ds(k * _L, _L)] for k in range(_RANK // _L)]
    bm = b_v[0, pl.ds(0, _L)]
    blv = b_v[1, pl.ds(0, _L)]
    lane = lax.iota(jnp.int32, _L)
    seven = jnp.full((_L,), 7, jnp.int32)
    zero = jnp.zeros((_L,), jnp.float32)

    def chunk_step(ci, _):
        c0 = ci * _CH
        for g in range(_CH // _L):
            uv = lax.shift_right_logical(uidx[pl.ds(c0 + g * _L, _L)], 3)
            iv = lax.shift_right_logical(iidx[pl.ds(c0 + g * _L, _L)], 3)
            for l in range(_L):
                slot = g * _L + l
                pltpu.async_copy(utab_hbm.at[uv[l]], ubuf.at[slot], sem)
                pltpu.async_copy(itab_hbm.at[iv[l]], ibuf.at[slot], sem)
        pltpu.make_async_copy(utab_hbm.at[pl.ds(0, _CH)], ubuf, sem).wait()
        pltpu.make_async_copy(itab_hbm.at[pl.ds(0, _CH)], ibuf, sem).wait()

        for g in range(_CH // _L):
            b0 = c0 + g * _L
            su = uidx[pl.ds(b0, _L)] & seven
            si = iidx[pl.ds(b0, _L)] & seven
            accm = zero
            acclv = zero
            for r in range(_L):
                slot = g * _L + r
                am = None
                alv = None
                for k in range(_RANK // _L):
                    u = ubuf[slot, su[r], pl.ds(k * _L, _L)]
                    it = ibuf[slot, si[r], pl.ds(k * _L, _L)]
                    inter = u * it
                    tm = inter * wm[k]
                    tlv = inter * wlv[k]
                    am = tm if am is None else am + tm
                    alv = tlv if alv is None else alv + tlv
                sel = lane == r
                accm = jnp.where(sel, jnp.sum(am), accm)
                acclv = jnp.where(sel, jnp.sum(alv), acclv)
            lv = acclv + blv
            mean_v[pl.ds(b0, _L)] = accm + bm
            logvar_v[pl.ds(b0, _L)] = lv
            std_v[pl.ds(b0, _L)] = jnp.exp(0.5 * lv)
        return _

    lax.fori_loop(0, _NCH, chunk_step, 0)

    pltpu.sync_copy(mean_v, mean_hbm.at[pl.ds(base, _BPW)])
    pltpu.sync_copy(std_v, std_hbm.at[pl.ds(base, _BPW)])
    pltpu.sync_copy(logvar_v, logvar_hbm.at[pl.ds(base, _BPW)])


@jax.jit
def _dpr(users, items, utab3, itab3, w_cat, bv):
    mesh = plsc.VectorSubcoreMesh(core_axis_name="c", subcore_axis_name="s")
    out = jax.ShapeDtypeStruct((_BATCH,), jnp.float32)
    f = pl.kernel(
        _dpr_body,
        out_type=(out, out, out),
        mesh=mesh,
        scratch_types=[
            pltpu.VMEM((_BPW,), jnp.int32),
            pltpu.VMEM((_BPW,), jnp.int32),
            pltpu.VMEM((_CH, 8, _RANK), jnp.float32),
            pltpu.VMEM((_CH, 8, _RANK), jnp.float32),
            pltpu.VMEM((_BPW,), jnp.float32),
            pltpu.VMEM((_BPW,), jnp.float32),
            pltpu.VMEM((_BPW,), jnp.float32),
            pltpu.VMEM((2, _RANK), jnp.float32),
            pltpu.VMEM((2, _L), jnp.float32),
            pltpu.SemaphoreType.DMA,
        ],
        compiler_params=pltpu.CompilerParams(needs_layout_passes=False),
    )
    return f(users, items, utab3, itab3, w_cat, bv)


def kernel(users, items, user_table, item_table, W_mean, b_mean, W_logvar,
           b_logvar):
    utab3 = user_table.reshape(-1, 8, _RANK)
    itab3 = item_table.reshape(-1, 8, _RANK)
    w_cat = jnp.stack([W_mean.reshape(_RANK), W_logvar.reshape(_RANK)])
    bv = jnp.stack([jnp.full((_L,), b_mean[0], jnp.float32),
                    jnp.full((_L,), b_logvar[0], jnp.float32)])
    mean, std, logvar = _dpr(users, items, utab3, itab3, w_cat, bv)
    return (mean, std, logvar)
